# baseline (speedup 1.0000x reference)
import functools
import math

import jax
import jax.numpy as jnp
from jax import lax
from jax.experimental import pallas as pl
from jax.experimental.pallas import tpu as pltpu
from jax.experimental.pallas import tpu_sc as plsc

D_MODEL = 1024
SCALE = math.sqrt(D_MODEL)

_INFO = plsc.get_sparse_core_info()
NC, NS, L = _INFO.num_cores, _INFO.num_subcores, _INFO.num_lanes
NW = NC * NS

CH = 16
NG = 4
NSB = 2


def _emb_body(b_per_w, n_chunk, x_hbm, table_hbm, out_hbm,
              idx_v, gbuf, sbuf, gsem, ssem):
    wid = lax.axis_index("s") * NC + lax.axis_index("c")
    base = wid * b_per_w

    pltpu.sync_copy(x_hbm.at[pl.ds(base, b_per_w)], idx_v)

    def gather(c, b):
        return pltpu.make_async_copy(
            table_hbm.at[idx_v.at[pl.ds(c * CH, CH)]],
            gbuf.at[b], gsem.at[b])

    def store(c, b):
        return pltpu.make_async_copy(
            sbuf.at[b], out_hbm.at[pl.ds(base + c * CH, CH)], ssem.at[b])

    for b in range(NG):
        gather(b, b).start()

    n_group = n_chunk // NG

    def group(g, _):
        for b in range(NG):
            c = g * NG + b
            sb = b % NSB
            gather(c, b).wait()

            @pl.when(c >= NSB)
            def _():
                store(c - NSB, sb).wait()

            def scale_row(r, _):
                for k in range(D_MODEL // L):
                    sbuf[sb, r, pl.ds(k * L, L)] = (
                        gbuf[b, r, pl.ds(k * L, L)] * SCALE)
                return 0

            lax.fori_loop(0, CH, scale_row, 0, unroll=False)
            store(c, sb).start()

            @pl.when(c + NG < n_chunk)
            def _():
                gather(c + NG, b).start()
        return 0

    lax.fori_loop(0, n_group, group, 0, unroll=False)
    for c in range(n_chunk - NSB, n_chunk):
        store(c, c % NSB).wait()


def kernel(x, table):
    orig_shape = x.shape
    xf = x.reshape(-1).astype(jnp.int32)
    b_total = xf.shape[0]
    b_per_w = b_total // NW
    n_chunk = b_per_w // CH

    mesh = plsc.VectorSubcoreMesh(core_axis_name="c", subcore_axis_name="s")
    k = pl.kernel(
        functools.partial(_emb_body, b_per_w, n_chunk),
        mesh=mesh,
        out_type=jax.ShapeDtypeStruct((b_total, D_MODEL), jnp.float32),
        scratch_types=[
            pltpu.VMEM((b_per_w,), jnp.int32),
            pltpu.VMEM((NG, CH, D_MODEL), jnp.float32),
            pltpu.VMEM((NSB, CH, D_MODEL), jnp.float32),
            pltpu.SemaphoreType.DMA((NG,)),
            pltpu.SemaphoreType.DMA((NSB,)),
        ],
    )
    out = k(xf, table)
    return out.reshape(*orig_shape, D_MODEL)


# speedup vs baseline: 1.3718x; 1.3718x over previous
import functools
import math

import jax
import jax.numpy as jnp
from jax import lax
from jax.experimental import pallas as pl
from jax.experimental.pallas import tpu as pltpu
from jax.experimental.pallas import tpu_sc as plsc

D_MODEL = 1024
SCALE = math.sqrt(D_MODEL)

_INFO = plsc.get_sparse_core_info()
NC, NS, L = _INFO.num_cores, _INFO.num_subcores, _INFO.num_lanes
NW = NC * NS

CH = 32
NBUF = 3


def _emb_body(b_per_w, n_chunk, x_hbm, table_hbm, out_hbm,
              idx_v, rows_v, sem):
    wid = lax.axis_index("s") * NC + lax.axis_index("c")
    base = wid * b_per_w

    pltpu.sync_copy(x_hbm.at[pl.ds(base, b_per_w)], idx_v)

    def gather(c):
        return pltpu.make_async_copy(
            table_hbm.at[idx_v.at[pl.ds(c * CH, CH)]],
            rows_v.at[c % NBUF], sem.at[c % NBUF])

    def store(c):
        return pltpu.make_async_copy(
            rows_v.at[c % NBUF],
            out_hbm.at[pl.ds(base + c * CH, CH)], sem.at[c % NBUF])

    gather(0).start()
    gather(1).start()
    for c in range(n_chunk):
        b = c % NBUF
        gather(c).wait()

        def scale_row(r, _, b=b):
            for k in range(D_MODEL // L):
                rows_v[b, r, pl.ds(k * L, L)] = (
                    rows_v[b, r, pl.ds(k * L, L)] * SCALE)
            return 0

        lax.fori_loop(0, CH, scale_row, 0, unroll=False)
        store(c).start()
        if c + 2 < n_chunk:
            if c >= 1:
                store(c - 1).wait()
            gather(c + 2).start()
    store(n_chunk - 2).wait()
    store(n_chunk - 1).wait()


def kernel(x, table):
    orig_shape = x.shape
    xf = x.reshape(-1).astype(jnp.int32)
    b_total = xf.shape[0]
    b_per_w = b_total // NW
    n_chunk = b_per_w // CH

    mesh = plsc.VectorSubcoreMesh(core_axis_name="c", subcore_axis_name="s")
    k = pl.kernel(
        functools.partial(_emb_body, b_per_w, n_chunk),
        mesh=mesh,
        out_type=jax.ShapeDtypeStruct((b_total, D_MODEL), jnp.float32),
        scratch_types=[
            pltpu.VMEM((b_per_w,), jnp.int32),
            pltpu.VMEM((NBUF, CH, D_MODEL), jnp.float32),
            pltpu.SemaphoreType.DMA((NBUF,)),
        ],
    )
    out = k(xf, table)
    return out.reshape(*orig_shape, D_MODEL)


# speedup vs baseline: 1.3791x; 1.0053x over previous
import functools
import math

import jax
import jax.numpy as jnp
from jax import lax
from jax.experimental import pallas as pl
from jax.experimental.pallas import tpu as pltpu
from jax.experimental.pallas import tpu_sc as plsc

D_MODEL = 1024
SCALE = math.sqrt(D_MODEL)

_INFO = plsc.get_sparse_core_info()
NC, NS, L = _INFO.num_cores, _INFO.num_subcores, _INFO.num_lanes
NW = NC * NS

CH = 32
NBUF = 3


def _emb_body(b_per_w, n_chunk, x_hbm, table_hbm, out_hbm,
              idx_v, rows_v, sem):
    wid = lax.axis_index("s") * NC + lax.axis_index("c")
    base = wid * b_per_w

    pltpu.sync_copy(x_hbm.at[pl.ds(base, b_per_w)], idx_v)

    def gather(c):
        return pltpu.make_async_copy(
            table_hbm.at[idx_v.at[pl.ds(c * CH, CH)]],
            rows_v.at[c % NBUF], sem.at[c % NBUF])

    def store(c):
        return pltpu.make_async_copy(
            rows_v.at[c % NBUF],
            out_hbm.at[pl.ds(base + c * CH, CH)], sem.at[c % NBUF])

    gather(0).start()
    gather(1).start()
    for c in range(n_chunk):
        b = c % NBUF
        gather(c).wait()

        def scale_row(r, _, b=b):
            for k in range(D_MODEL // L):
                rows_v[b, r, pl.ds(k * L, L)] = (
                    rows_v[b, r, pl.ds(k * L, L)] * SCALE)
            return 0

        lax.fori_loop(0, CH, scale_row, 0, unroll=False)
        store(c).start()
        if c + 2 < n_chunk:
            if c >= 1:
                store(c - 1).wait()
            gather(c + 2).start()
    for c in range(n_chunk - NBUF, n_chunk):
        store(c).wait()


def kernel(x, table):
    orig_shape = x.shape
    xf = x.reshape(-1).astype(jnp.int32)
    b_total = xf.shape[0]
    b_per_w = b_total // NW
    n_chunk = b_per_w // CH

    mesh = plsc.VectorSubcoreMesh(core_axis_name="c", subcore_axis_name="s")
    k = pl.kernel(
        functools.partial(_emb_body, b_per_w, n_chunk),
        mesh=mesh,
        out_type=jax.ShapeDtypeStruct((b_total, D_MODEL), jnp.float32),
        scratch_types=[
            pltpu.VMEM((b_per_w,), jnp.int32),
            pltpu.VMEM((NBUF, CH, D_MODEL), jnp.float32),
            pltpu.SemaphoreType.DMA((NBUF,)),
        ],
    )
    out = k(xf, table)
    return out.reshape(*orig_shape, D_MODEL)
